# baseline (speedup 1.0000x reference)
import functools

import jax
import jax.numpy as jnp
from jax import lax
from jax.experimental import pallas as pl
from jax.experimental.pallas import tpu as pltpu
from jax.experimental.pallas import tpu_sc as plsc

EMBED = 64
BATCH = 16384
L = 16
NC, NS = 2, 16
NW = NC * NS
BPW = BATCH // NW
CHUNK = 128
NCHUNK = BPW // CHUNK
NGRP = BPW // L
CPR = EMBED // L


def _make_sc_kernel():
  mesh = plsc.VectorSubcoreMesh(core_axis_name="c", subcore_axis_name="s")

  @functools.partial(
      pl.kernel,
      mesh=mesh,
      out_type=jax.ShapeDtypeStruct((BATCH,), jnp.float32),
      compiler_params=pltpu.CompilerParams(
          needs_layout_passes=False, use_tc_tiling_on_sc=False),
      scratch_types=[
          pltpu.VMEM((NCHUNK, CHUNK), jnp.int32),
          pltpu.VMEM((NCHUNK, CHUNK), jnp.int32),
          pltpu.VMEM((BPW, EMBED), jnp.float32),
          pltpu.VMEM((BPW, EMBED), jnp.float32),
          pltpu.VMEM((L, L), jnp.float32),
          pltpu.VMEM((BPW,), jnp.float32),
          pltpu.SemaphoreType.DMA,
      ],
  )
  def sc_kernel(uidx_hbm, iidx_hbm, utab_hbm, itab_hbm, out_hbm,
                uidx_v, iidx_v, urows_v, irows_v, s_v, out_v, sem):
    wid = lax.axis_index("s") * NC + lax.axis_index("c")
    base = wid * BPW

    pltpu.sync_copy(uidx_hbm.at[pl.ds(wid * NCHUNK, NCHUNK)], uidx_v)
    pltpu.sync_copy(iidx_hbm.at[pl.ds(wid * NCHUNK, NCHUNK)], iidx_v)

    copies = []
    for j in range(NCHUNK):
      copies.append(pltpu.async_copy(
          utab_hbm.at[uidx_v.at[j]],
          urows_v.at[pl.ds(j * CHUNK, CHUNK)], sem))
      copies.append(pltpu.async_copy(
          itab_hbm.at[iidx_v.at[j]],
          irows_v.at[pl.ds(j * CHUNK, CHUNK)], sem))
    for cp in copies:
      cp.wait()

    lane = lax.iota(jnp.int32, L)

    def group_body(g, carry):
      rbase = g * L
      for i in range(L):
        r = rbase + i
        acc = urows_v[r, pl.ds(0, L)] * irows_v[r, pl.ds(0, L)]
        for c in range(1, CPR):
          acc = acc + (urows_v[r, pl.ds(c * L, L)] *
                       irows_v[r, pl.ds(c * L, L)])
        s_v[i, :] = acc
      tot = plsc.load_gather(s_v, [lane, jnp.zeros((L,), jnp.int32)])
      for c in range(1, L):
        tot = tot + plsc.load_gather(
            s_v, [lane, jnp.full((L,), c, jnp.int32)])
      out_v[pl.ds(rbase, L)] = tot
      return carry

    lax.fori_loop(0, NGRP, group_body, 0)
    pltpu.sync_copy(out_v, out_hbm.at[pl.ds(base, BPW)])

  return sc_kernel


_SC_KERNEL = _make_sc_kernel()


def kernel(user_indices, item_indices, user_table, item_table):
  uidx = user_indices.astype(jnp.int32).reshape(NW * NCHUNK, CHUNK)
  iidx = item_indices.astype(jnp.int32).reshape(NW * NCHUNK, CHUNK)
  return _SC_KERNEL(uidx, iidx, user_table, item_table)
